# baseline (speedup 1.0000x reference)
import jax
import jax.numpy as jnp
from jax import lax
from jax.experimental import pallas as pl
from jax.experimental.pallas import tpu as pltpu
from jax.experimental.pallas import tpu_sc as plsc

_SEQLEN = 2048
_EMBED = 768
_BATCH = 4

_NUM_WORKERS = 32
_TOKENS = _BATCH * _SEQLEN
_POSW = _SEQLEN // _NUM_WORKERS
_CHUNK = 32
_NCHUNKS = _BATCH * (_POSW // _CHUNK)
_LANES = 16


def _emb_body(ids_hbm, wte_hbm, wpe_hbm, out_hbm,
              pos_v, idx0, idx1, rows0, rows1, gsem0, gsem1, ssem0, ssem1):
    wid = lax.axis_index("s") * 2 + lax.axis_index("c")
    pbase = wid * _POSW
    idxs = (idx0, idx1)
    rows = (rows0, rows1)
    gsems = (gsem0, gsem1)
    ssems = (ssem0, ssem1)

    pltpu.sync_copy(wpe_hbm.at[pl.ds(pbase, _POSW)], pos_v)

    def flat_off(t):
        b, h = divmod(t, _POSW // _CHUNK)
        return b * _SEQLEN + pbase + h * _CHUNK, h * _CHUNK

    def start_gather(t):
        nb = t % 2
        off, _ = flat_off(t)
        pltpu.sync_copy(ids_hbm.at[pl.ds(off, _CHUNK)], idxs[nb])
        return pltpu.async_copy(wte_hbm.at[idxs[nb]], rows[nb], gsems[nb])

    gathers = [None] * _NCHUNKS
    stores = [None] * _NCHUNKS
    gathers[0] = start_gather(0)
    for t in range(_NCHUNKS):
        buf = t % 2
        if t + 1 < _NCHUNKS:
            if t >= 1:
                stores[t - 1].wait()
            gathers[t + 1] = start_gather(t + 1)
        gathers[t].wait()
        off, poff = flat_off(t)

        def add_row(r, carry):
            for k in range(_EMBED // _LANES):
                sl = pl.ds(k * _LANES, _LANES)
                rows[buf][r, sl] = rows[buf][r, sl] + pos_v[poff + r, sl]
            return carry

        lax.fori_loop(0, _CHUNK, add_row, 0)
        stores[t] = pltpu.async_copy(rows[buf], out_hbm.at[pl.ds(off, _CHUNK)],
                                     ssems[buf])
    stores[_NCHUNKS - 2].wait()
    stores[_NCHUNKS - 1].wait()


@jax.jit
def kernel(input_ids, token_embeddings, position_embeddings):
    ids_flat = input_ids.reshape(_TOKENS)
    mesh = plsc.VectorSubcoreMesh(core_axis_name="c", subcore_axis_name="s")
    out = pl.kernel(
        _emb_body,
        out_type=jax.ShapeDtypeStruct((_TOKENS, _EMBED), jnp.float32),
        mesh=mesh,
        scratch_types=[
            pltpu.VMEM((_POSW, _EMBED), jnp.float32),
            pltpu.VMEM((_CHUNK,), jnp.int32),
            pltpu.VMEM((_CHUNK,), jnp.int32),
            pltpu.VMEM((_CHUNK, _EMBED), jnp.float32),
            pltpu.VMEM((_CHUNK, _EMBED), jnp.float32),
            pltpu.SemaphoreType.DMA,
            pltpu.SemaphoreType.DMA,
            pltpu.SemaphoreType.DMA,
            pltpu.SemaphoreType.DMA,
        ],
    )(ids_flat, token_embeddings, position_embeddings)
    return out.reshape(_BATCH, _SEQLEN, _EMBED)


# speedup vs baseline: 1.2953x; 1.2953x over previous
import jax
import jax.numpy as jnp
from jax import lax
from jax.experimental import pallas as pl
from jax.experimental.pallas import tpu as pltpu
from jax.experimental.pallas import tpu_sc as plsc

_SEQLEN = 2048
_EMBED = 768
_BATCH = 4

_NUM_WORKERS = 32
_TOKENS = _BATCH * _SEQLEN
_POSW = _SEQLEN // _NUM_WORKERS
_CHUNK = 32
_NCHUNKS = _BATCH * (_POSW // _CHUNK)
_NBUF = 3
_LANES = 16


def _emb_body(ids_hbm, wte_hbm, wpe_hbm, out_hbm,
              idx_v, pos_v, rows0, rows1, rows2,
              psem, gsem0, gsem1, gsem2, ssem0, ssem1, ssem2):
    wid = lax.axis_index("s") * 2 + lax.axis_index("c")
    pbase = wid * _POSW
    rows = (rows0, rows1, rows2)
    gsems = (gsem0, gsem1, gsem2)
    ssems = (ssem0, ssem1, ssem2)

    def flat_off(t):
        b, h = divmod(t, _POSW // _CHUNK)
        return b * _SEQLEN + pbase + h * _CHUNK, h * _CHUNK

    pos_cp = pltpu.async_copy(wpe_hbm.at[pl.ds(pbase, _POSW)], pos_v, psem)
    for b in range(_BATCH):
        pltpu.sync_copy(ids_hbm.at[pl.ds(b * _SEQLEN + pbase, _POSW)],
                        idx_v.at[pl.ds(b * _POSW, _POSW)])

    gads = [None] * _NCHUNKS
    stores = [None] * _NCHUNKS
    for step in range(_NCHUNKS + 1):
        if step < _NCHUNKS:
            q = step % _NBUF
            if step >= _NBUF:
                stores[step - _NBUF].wait()
            b, h = divmod(step, _POSW // _CHUNK)
            isl = pl.ds(b * _POSW + h * _CHUNK, _CHUNK)
            gads[step] = pltpu.async_copy(
                wte_hbm.at[idx_v.at[isl]], rows[q], gsems[q])
        u = step - 1
        if 0 <= u < _NCHUNKS:
            q = u % _NBUF
            gads[u].wait()
            if u == 0:
                pos_cp.wait()
            off, poff = flat_off(u)

            def add_row(r, carry):
                for k in range(_EMBED // _LANES):
                    sl = pl.ds(k * _LANES, _LANES)
                    plsc.addupdate(rows[q].at[r, sl], pos_v[poff + r, sl])
                return carry

            lax.fori_loop(0, _CHUNK, add_row, 0)
            stores[u] = pltpu.async_copy(
                rows[q], out_hbm.at[pl.ds(off, _CHUNK)], ssems[q])
    for u in range(_NCHUNKS - _NBUF, _NCHUNKS):
        stores[u].wait()


@jax.jit
def kernel(input_ids, token_embeddings, position_embeddings):
    ids_flat = input_ids.reshape(_TOKENS)
    mesh = plsc.VectorSubcoreMesh(core_axis_name="c", subcore_axis_name="s")
    out = pl.kernel(
        _emb_body,
        out_type=jax.ShapeDtypeStruct((_TOKENS, _EMBED), jnp.float32),
        mesh=mesh,
        scratch_types=(
            [pltpu.VMEM((_BATCH * _POSW,), jnp.int32),
             pltpu.VMEM((_POSW, _EMBED), jnp.float32)]
            + [pltpu.VMEM((_CHUNK, _EMBED), jnp.float32)] * _NBUF
            + [pltpu.SemaphoreType.DMA] * (1 + 2 * _NBUF)
        ),
    )(ids_flat, token_embeddings, position_embeddings)
    return out.reshape(_BATCH, _SEQLEN, _EMBED)


# speedup vs baseline: 1.3345x; 1.0303x over previous
import jax
import jax.numpy as jnp
from jax import lax
from jax.experimental import pallas as pl
from jax.experimental.pallas import tpu as pltpu
from jax.experimental.pallas import tpu_sc as plsc

_SEQLEN = 2048
_EMBED = 768
_BATCH = 4

_NUM_WORKERS = 32
_TOKENS = _BATCH * _SEQLEN
_POSW = _SEQLEN // _NUM_WORKERS
_CHUNK = 32
_NCHUNKS = _BATCH * (_POSW // _CHUNK)
_NBUF = 3
_LANES = 16


def _emb_body(ids_hbm, wte_hbm, wpe_hbm, out_hbm,
              idx_v, pos_v, rows0, rows1, rows2,
              psem, isem, gsem0, gsem1, gsem2, ssem0, ssem1, ssem2):
    wid = lax.axis_index("s") * 2 + lax.axis_index("c")
    pbase = wid * _POSW
    rows = (rows0, rows1, rows2)
    gsems = (gsem0, gsem1, gsem2)
    ssems = (ssem0, ssem1, ssem2)

    def flat_off(t):
        b, h = divmod(t, _POSW // _CHUNK)
        return b * _SEQLEN + pbase + h * _CHUNK, h * _CHUNK

    pos_cp = pltpu.async_copy(wpe_hbm.at[pl.ds(pbase, _POSW)], pos_v, psem)
    idx_cps = []
    for b in range(_BATCH):
        idx_cps.append(pltpu.async_copy(
            ids_hbm.at[pl.ds(b * _SEQLEN + pbase, _POSW)],
            idx_v.at[pl.ds(b * _POSW, _POSW)], isem))

    gads = [None] * _NCHUNKS
    stores = [None] * _NCHUNKS
    idx_ready = 0
    for step in range(_NCHUNKS + 1):
        if step < _NCHUNKS:
            q = step % _NBUF
            if step >= _NBUF:
                stores[step - _NBUF].wait()
            b, h = divmod(step, _POSW // _CHUNK)
            while idx_ready <= b:
                idx_cps[idx_ready].wait()
                idx_ready += 1
            isl = pl.ds(b * _POSW + h * _CHUNK, _CHUNK)
            gads[step] = pltpu.async_copy(
                wte_hbm.at[idx_v.at[isl]], rows[q], gsems[q])
        u = step - 1
        if 0 <= u < _NCHUNKS:
            q = u % _NBUF
            gads[u].wait()
            if u == 0:
                pos_cp.wait()
            off, poff = flat_off(u)

            @plsc.parallel_loop(0, _CHUNK, 1, unroll=4)
            def add_row(r):
                for k in range(_EMBED // _LANES):
                    sl = pl.ds(k * _LANES, _LANES)
                    plsc.addupdate(rows[q].at[r, sl], pos_v[poff + r, sl])

            stores[u] = pltpu.async_copy(
                rows[q], out_hbm.at[pl.ds(off, _CHUNK)], ssems[q])
    for u in range(_NCHUNKS - _NBUF, _NCHUNKS):
        stores[u].wait()


@jax.jit
def kernel(input_ids, token_embeddings, position_embeddings):
    ids_flat = input_ids.reshape(_TOKENS)
    mesh = plsc.VectorSubcoreMesh(core_axis_name="c", subcore_axis_name="s")
    out = pl.kernel(
        _emb_body,
        out_type=jax.ShapeDtypeStruct((_TOKENS, _EMBED), jnp.float32),
        mesh=mesh,
        scratch_types=(
            [pltpu.VMEM((_BATCH * _POSW,), jnp.int32),
             pltpu.VMEM((_POSW, _EMBED), jnp.float32)]
            + [pltpu.VMEM((_CHUNK, _EMBED), jnp.float32)] * _NBUF
            + [pltpu.SemaphoreType.DMA] * (2 + 2 * _NBUF)
        ),
    )(ids_flat, token_embeddings, position_embeddings)
    return out.reshape(_BATCH, _SEQLEN, _EMBED)


# speedup vs baseline: 1.5112x; 1.1324x over previous
import jax
import jax.numpy as jnp
from jax import lax
from jax.experimental import pallas as pl
from jax.experimental.pallas import tpu as pltpu
from jax.experimental.pallas import tpu_sc as plsc

_SEQLEN = 2048
_EMBED = 768
_BATCH = 4

_NUM_WORKERS = 32
_TOKENS = _BATCH * _SEQLEN
_POSW = _SEQLEN // _NUM_WORKERS
_GROUP = 16
_NGROUPS = _POSW // _GROUP
_LANES = 16


def _emb_body(ids_hbm, wte_hbm, wpe_hbm, out_hbm,
              idx_v, pos0, pos1,
              r00, r01, r02, r03, r10, r11, r12, r13,
              isem, psem0, psem1,
              gsem0, gsem1, ssem0, ssem1):
    wid = lax.axis_index("s") * 2 + lax.axis_index("c")
    pbase = wid * _POSW
    pos = (pos0, pos1)
    rows = ((r00, r01, r02, r03), (r10, r11, r12, r13))
    psems = (psem0, psem1)
    gsems = (gsem0, gsem1)
    ssems = (ssem0, ssem1)

    idx_cps = [pltpu.async_copy(
        ids_hbm.at[pl.ds(b * _SEQLEN + pbase, _POSW)],
        idx_v.at[pl.ds(b * _POSW, _POSW)], isem) for b in range(_BATCH)]
    for cp in idx_cps:
        cp.wait()

    poscps = [None] * _NGROUPS
    gads = [[None] * _BATCH for _ in range(_NGROUPS)]
    stores = [[None] * _BATCH for _ in range(_NGROUPS)]
    for step in range(_NGROUPS + 1):
        if step < _NGROUPS:
            q = step % 2
            if step >= 2:
                for cp in stores[step - 2]:
                    cp.wait()
            poscps[step] = pltpu.async_copy(
                wpe_hbm.at[pl.ds(pbase + step * _GROUP, _GROUP)],
                pos[q], psems[q])
            for b in range(_BATCH):
                isl = pl.ds(b * _POSW + step * _GROUP, _GROUP)
                gads[step][b] = pltpu.async_copy(
                    wte_hbm.at[idx_v.at[isl]], rows[q][b], gsems[q])
        u = step - 1
        if 0 <= u < _NGROUPS:
            q = u % 2
            poscps[u].wait()
            for b in range(_BATCH):
                gads[u][b].wait()

            @plsc.parallel_loop(0, _GROUP, 1, unroll=2)
            def add_row(r):
                for k in range(_EMBED // _LANES):
                    sl = pl.ds(k * _LANES, _LANES)
                    p = pos[q][r, sl]
                    for b in range(_BATCH):
                        rows[q][b][r, sl] = rows[q][b][r, sl] + p

            for b in range(_BATCH):
                off = b * _SEQLEN + pbase + u * _GROUP
                stores[u][b] = pltpu.async_copy(
                    rows[q][b], out_hbm.at[pl.ds(off, _GROUP)], ssems[q])
    for u in (_NGROUPS - 2, _NGROUPS - 1):
        for cp in stores[u]:
            cp.wait()


@jax.jit
def kernel(input_ids, token_embeddings, position_embeddings):
    ids_flat = input_ids.reshape(_TOKENS)
    mesh = plsc.VectorSubcoreMesh(core_axis_name="c", subcore_axis_name="s")
    out = pl.kernel(
        _emb_body,
        out_type=jax.ShapeDtypeStruct((_TOKENS, _EMBED), jnp.float32),
        mesh=mesh,
        scratch_types=(
            [pltpu.VMEM((_BATCH * _POSW,), jnp.int32)]
            + [pltpu.VMEM((_GROUP, _EMBED), jnp.float32)] * 2
            + [pltpu.VMEM((_GROUP, _EMBED), jnp.float32)] * (2 * _BATCH)
            + [pltpu.SemaphoreType.DMA] * 7
        ),
    )(ids_flat, token_embeddings, position_embeddings)
    return out.reshape(_BATCH, _SEQLEN, _EMBED)


# speedup vs baseline: 1.5196x; 1.0055x over previous
import jax
import jax.numpy as jnp
from jax import lax
from jax.experimental import pallas as pl
from jax.experimental.pallas import tpu as pltpu
from jax.experimental.pallas import tpu_sc as plsc

_SEQLEN = 2048
_EMBED = 768
_BATCH = 4

_NUM_WORKERS = 32
_TOKENS = _BATCH * _SEQLEN
_POSW = _SEQLEN // _NUM_WORKERS
_GROUP = 16
_NGROUPS = _POSW // _GROUP
_LANES = 16


def _emb_body(ids_hbm, wte_hbm, wpe_hbm, out_hbm,
              idx_v, pos0, pos1,
              r00, r01, r02, r03, r10, r11, r12, r13,
              isem, psem0, psem1,
              gsem0, gsem1, ssem0, ssem1):
    wid = lax.axis_index("s") * 2 + lax.axis_index("c")
    pbase = wid * _POSW
    pos = (pos0, pos1)
    rows = ((r00, r01, r02, r03), (r10, r11, r12, r13))
    psems = (psem0, psem1)
    gsems = (gsem0, gsem1)
    ssems = (ssem0, ssem1)

    idx_cps = [pltpu.async_copy(
        ids_hbm.at[b, pl.ds(pbase, _POSW)],
        idx_v.at[pl.ds(b * _POSW, _POSW)], isem) for b in range(_BATCH)]
    for cp in idx_cps:
        cp.wait()

    poscps = [None] * _NGROUPS
    gads = [[None] * _BATCH for _ in range(_NGROUPS)]
    stores = [[None] * _BATCH for _ in range(_NGROUPS)]
    for step in range(_NGROUPS + 1):
        if step < _NGROUPS:
            q = step % 2
            if step >= 2:
                for cp in stores[step - 2]:
                    cp.wait()
            poscps[step] = pltpu.async_copy(
                wpe_hbm.at[pl.ds(pbase + step * _GROUP, _GROUP)],
                pos[q], psems[q])
            for b in range(_BATCH):
                isl = pl.ds(b * _POSW + step * _GROUP, _GROUP)
                gads[step][b] = pltpu.async_copy(
                    wte_hbm.at[idx_v.at[isl]], rows[q][b], gsems[q])
        u = step - 1
        if 0 <= u < _NGROUPS:
            q = u % 2
            poscps[u].wait()
            for b in range(_BATCH):
                gads[u][b].wait()

            @plsc.parallel_loop(0, _GROUP, 1, unroll=2)
            def add_row(r):
                for k in range(_EMBED // _LANES):
                    sl = pl.ds(k * _LANES, _LANES)
                    p = pos[q][r, sl]
                    for b in range(_BATCH):
                        rows[q][b][r, sl] = rows[q][b][r, sl] + p

            for b in range(_BATCH):
                off = pbase + u * _GROUP
                stores[u][b] = pltpu.async_copy(
                    rows[q][b], out_hbm.at[b, pl.ds(off, _GROUP)], ssems[q])
    for u in (_NGROUPS - 2, _NGROUPS - 1):
        for cp in stores[u]:
            cp.wait()


@jax.jit
def kernel(input_ids, token_embeddings, position_embeddings):
    mesh = plsc.VectorSubcoreMesh(core_axis_name="c", subcore_axis_name="s")
    out = pl.kernel(
        _emb_body,
        out_type=jax.ShapeDtypeStruct((_BATCH, _SEQLEN, _EMBED), jnp.float32),
        mesh=mesh,
        scratch_types=(
            [pltpu.VMEM((_BATCH * _POSW,), jnp.int32)]
            + [pltpu.VMEM((_GROUP, _EMBED), jnp.float32)] * 2
            + [pltpu.VMEM((_GROUP, _EMBED), jnp.float32)] * (2 * _BATCH)
            + [pltpu.SemaphoreType.DMA] * 7
        ),
    )(input_ids, token_embeddings, position_embeddings)
    return out
